# baseline (speedup 1.0000x reference)
import functools

import jax
import jax.numpy as jnp
from jax import lax
from jax.experimental import pallas as pl
from jax.experimental.pallas import tpu as pltpu
from jax.experimental.pallas import tpu_sc as plsc

VOCAB = 1000000
EMBED = 128
BATCH = 16384

_NC = 2
_NS = 16
_NW = _NC * _NS
_CH = 128


def _sc_gather_body(nch, idx_a_hbm, idx_b_hbm, table_hbm, out_ab,
                    idx_v, rows_v, *sems):
    bpw = nch * _CH
    wid = lax.axis_index("s") * _NC + lax.axis_index("c")
    base = wid * bpw
    ia = pltpu.async_copy(idx_a_hbm.at[pl.ds(base, bpw)], idx_v.at[0], sems[0])
    ib = pltpu.async_copy(idx_b_hbm.at[pl.ds(base, bpw)], idx_v.at[1], sems[1])
    ia.wait()

    chunks = [(l, c) for l in range(2) for c in range(nch)]
    depth = 7
    lag = 4
    n = len(chunks)
    gathers = [None] * n
    stores = [None] * n
    for k in range(n + lag):
        if k >= depth:
            stores[k - depth].wait()
        if k < n:
            l, c = chunks[k]
            if (l, c) == (1, 0):
                ib.wait()
            gathers[k] = pltpu.async_copy(
                table_hbm.at[idx_v.at[l, pl.ds(c * _CH, _CH)]],
                rows_v.at[k % depth],
                sems[k % depth])
        j = k - lag
        if j >= 0:
            l, c = chunks[j]
            gathers[j].wait()
            stores[j] = pltpu.async_copy(
                rows_v.at[j % depth],
                out_ab.at[l, pl.ds(base + c * _CH, _CH)],
                sems[j % depth])
    for j in range(max(0, n - depth + lag), n):
        stores[j].wait()


def _sc_gather(idx_a, idx_b, emb):
    nrows = idx_a.shape[0]
    bpw = nrows // _NW
    nch = bpw // _CH
    mesh = plsc.VectorSubcoreMesh(core_axis_name="c", subcore_axis_name="s")
    fn = functools.partial(
        pl.kernel,
        mesh=mesh,
        out_type=jax.ShapeDtypeStruct((2, nrows, EMBED), jnp.float32),
        scratch_types=[
            pltpu.VMEM((2, bpw), jnp.int32),
            pltpu.VMEM((7, _CH, EMBED), jnp.float32),
            pltpu.SemaphoreType.DMA,
            pltpu.SemaphoreType.DMA,
            pltpu.SemaphoreType.DMA,
            pltpu.SemaphoreType.DMA,
            pltpu.SemaphoreType.DMA,
            pltpu.SemaphoreType.DMA,
            pltpu.SemaphoreType.DMA,
        ],
    )(functools.partial(_sc_gather_body, nch))
    return fn(idx_a, idx_b, emb)


_BB = 8192


def _mlp_body(x_ref, w1_ref, b1_ref, w2_ref, b2_ref, out_ref):
    ea = x_ref[0]
    eb = x_ref[1]
    h = lax.dot_general(ea, w1_ref[:, :EMBED], (((1,), (1,)), ((), ())),
                        preferred_element_type=jnp.float32)
    h = h + lax.dot_general(eb, w1_ref[:, EMBED:], (((1,), (1,)), ((), ())),
                            preferred_element_type=jnp.float32)
    h = h + b1_ref[...]
    h = jnp.maximum(h, 0.0)
    out = lax.dot_general(w2_ref[...], h, (((1,), (1,)), ((), ())),
                          preferred_element_type=jnp.float32)
    out_ref[0, 0, :] = out[0, :] + b2_ref[0]


def _mlp(xab, W1, b1, W2, b2):
    nrows = xab.shape[1]
    nb = nrows // _BB
    out2d = pl.pallas_call(
        _mlp_body,
        grid=(nb,),
        in_specs=[
            pl.BlockSpec((2, _BB, EMBED), lambda i: (0, i, 0)),
            pl.BlockSpec((EMBED, 2 * EMBED), lambda i: (0, 0)),
            pl.BlockSpec((1, EMBED), lambda i: (0, 0)),
            pl.BlockSpec((1, EMBED), lambda i: (0, 0)),
            pl.BlockSpec(memory_space=pltpu.SMEM),
        ],
        out_specs=pl.BlockSpec((1, 1, _BB), lambda i: (i, 0, 0)),
        out_shape=jax.ShapeDtypeStruct((nb, 1, _BB), jnp.float32),
    )(xab, W1, b1.reshape(1, EMBED), W2, b2)
    return out2d.reshape(nrows)


def kernel(idx_a, idx_b, emb, W1, b1, W2, b2):
    xab = _sc_gather(idx_a, idx_b, emb)
    return _mlp(xab, W1, b1, W2, b2)


# speedup vs baseline: 1.0076x; 1.0076x over previous
import functools

import jax
import jax.numpy as jnp
from jax import lax
from jax.experimental import pallas as pl
from jax.experimental.pallas import tpu as pltpu
from jax.experimental.pallas import tpu_sc as plsc

VOCAB = 1000000
EMBED = 128
BATCH = 16384

_NC = 2
_NS = 16
_NW = _NC * _NS
_CH = 128


def _sc_gather_body(nch, idx_a_hbm, idx_b_hbm, table_hbm, out_a, out_b,
                    idx_v, rows_v, *sems):
    bpw = nch * _CH
    wid = lax.axis_index("s") * _NC + lax.axis_index("c")
    base = wid * bpw
    ia = pltpu.async_copy(idx_a_hbm.at[pl.ds(base, bpw)], idx_v.at[0], sems[0])
    ib = pltpu.async_copy(idx_b_hbm.at[pl.ds(base, bpw)], idx_v.at[1], sems[1])
    ia.wait()

    chunks = [(l, c) for l in range(2) for c in range(nch)]
    outs = (out_a, out_b)
    depth = 7
    lag = 4
    n = len(chunks)
    gathers = [None] * n
    stores = [None] * n
    for k in range(n + lag):
        if k >= depth:
            stores[k - depth].wait()
        if k < n:
            l, c = chunks[k]
            if (l, c) == (1, 0):
                ib.wait()
            gathers[k] = pltpu.async_copy(
                table_hbm.at[idx_v.at[l, pl.ds(c * _CH, _CH)]],
                rows_v.at[k % depth],
                sems[k % depth])
        j = k - lag
        if j >= 0:
            l, c = chunks[j]
            gathers[j].wait()
            stores[j] = pltpu.async_copy(
                rows_v.at[j % depth],
                outs[l].at[pl.ds(base + c * _CH, _CH)],
                sems[j % depth])
    for j in range(max(0, n - depth + lag), n):
        stores[j].wait()


def _sc_gather(idx_a, idx_b, emb):
    nrows = idx_a.shape[0]
    bpw = nrows // _NW
    nch = bpw // _CH
    mesh = plsc.VectorSubcoreMesh(core_axis_name="c", subcore_axis_name="s")
    fn = functools.partial(
        pl.kernel,
        mesh=mesh,
        out_type=[
            jax.ShapeDtypeStruct((nrows, EMBED), jnp.float32),
            jax.ShapeDtypeStruct((nrows, EMBED), jnp.float32),
        ],
        scratch_types=[
            pltpu.VMEM((2, bpw), jnp.int32),
            pltpu.VMEM((7, _CH, EMBED), jnp.float32),
            pltpu.SemaphoreType.DMA,
            pltpu.SemaphoreType.DMA,
            pltpu.SemaphoreType.DMA,
            pltpu.SemaphoreType.DMA,
            pltpu.SemaphoreType.DMA,
            pltpu.SemaphoreType.DMA,
            pltpu.SemaphoreType.DMA,
        ],
    )(functools.partial(_sc_gather_body, nch))
    return fn(idx_a, idx_b, emb)


_BB = 8192


def _mlp_body(ea_ref, eb_ref, w1_ref, b1_ref, w2_ref, b2_ref, out_ref):
    ea = ea_ref[...]
    eb = eb_ref[...]
    h = lax.dot_general(ea, w1_ref[:, :EMBED], (((1,), (1,)), ((), ())),
                        preferred_element_type=jnp.float32)
    h = h + lax.dot_general(eb, w1_ref[:, EMBED:], (((1,), (1,)), ((), ())),
                            preferred_element_type=jnp.float32)
    h = h + b1_ref[...]
    h = jnp.maximum(h, 0.0)
    out = lax.dot_general(w2_ref[...], h, (((1,), (1,)), ((), ())),
                          preferred_element_type=jnp.float32)
    out_ref[0, 0, :] = out[0, :] + b2_ref[0]


def _mlp(ea, eb, W1, b1, W2, b2):
    nrows = ea.shape[0]
    nb = nrows // _BB
    out2d = pl.pallas_call(
        _mlp_body,
        grid=(nb,),
        in_specs=[
            pl.BlockSpec((_BB, EMBED), lambda i: (i, 0)),
            pl.BlockSpec((_BB, EMBED), lambda i: (i, 0)),
            pl.BlockSpec((EMBED, 2 * EMBED), lambda i: (0, 0)),
            pl.BlockSpec((1, EMBED), lambda i: (0, 0)),
            pl.BlockSpec((1, EMBED), lambda i: (0, 0)),
            pl.BlockSpec(memory_space=pltpu.SMEM),
        ],
        out_specs=pl.BlockSpec((1, 1, _BB), lambda i: (i, 0, 0)),
        out_shape=jax.ShapeDtypeStruct((nb, 1, _BB), jnp.float32),
    )(ea, eb, W1, b1.reshape(1, EMBED), W2, b2)
    return out2d.reshape(nrows)


def kernel(idx_a, idx_b, emb, W1, b1, W2, b2):
    ea, eb = _sc_gather(idx_a, idx_b, emb)
    return _mlp(ea, eb, W1, b1, W2, b2)
